# Initial kernel scaffold; baseline (speedup 1.0000x reference)
import jax
import jax.numpy as jnp
from jax.experimental import pallas as pl


def kernel(tokens, table, ctx, prefix):
    raise NotImplementedError("write your pallas kernel here")


# speedup vs baseline: 1.1590x; 1.1590x over previous
import functools

import jax
import jax.numpy as jnp
from jax import lax
from jax.experimental import pallas as pl
from jax.experimental.pallas import tpu as pltpu
from jax.experimental.pallas import tpu_sc as plsc

N_CTX = 16
VOCAB = 49408
D = 512
B = 1024
L = 60
P = 1 + N_CTX + L

NC = 2
NS = 16
NW = NC * NS
NB = B // NW


def _sc_body(tokens_hbm, table_hbm, ctx_hbm, prefix_hbm,
             out_hbm,
             tok_v, buf0, buf1,
             gsem0, gsem1, wsem0, wsem1):
    wid = lax.axis_index("c") * NS + lax.axis_index("s")
    base = wid * NB

    pltpu.sync_copy(tokens_hbm.at[pl.ds(base, NB)], tok_v)

    for buf in (buf0, buf1):
        pltpu.sync_copy(prefix_hbm, buf.at[pl.ds(0, 1)])
        pltpu.sync_copy(ctx_hbm, buf.at[pl.ds(1, N_CTX)])

    bufs = (buf0, buf1)
    gsems = (gsem0, gsem1)
    wsems = (wsem0, wsem1)

    gathers = [None] * NB
    writes = [None] * NB
    gathers[0] = pltpu.async_copy(
        table_hbm.at[tok_v.at[0]], buf0.at[pl.ds(1 + N_CTX, L)], gsem0)
    for i in range(NB):
        p = i % 2
        q = (i + 1) % 2
        if i + 1 < NB:
            if i >= 1:
                writes[i - 1].wait()
            gathers[i + 1] = pltpu.async_copy(
                table_hbm.at[tok_v.at[i + 1]],
                bufs[q].at[pl.ds(1 + N_CTX, L)], gsems[q])
        gathers[i].wait()
        writes[i] = pltpu.async_copy(bufs[p], out_hbm.at[base + i], wsems[p])
    writes[NB - 2].wait()
    writes[NB - 1].wait()


_sc_call = functools.partial(
    pl.kernel,
    out_type=jax.ShapeDtypeStruct((B, P, D), jnp.float32),
    mesh=plsc.VectorSubcoreMesh(core_axis_name="c", subcore_axis_name="s"),
    scratch_types=[
        pltpu.VMEM((NB, L), jnp.int32),
        pltpu.VMEM((P, D), jnp.float32),
        pltpu.VMEM((P, D), jnp.float32),
        pltpu.SemaphoreType.DMA,
        pltpu.SemaphoreType.DMA,
        pltpu.SemaphoreType.DMA,
        pltpu.SemaphoreType.DMA,
    ],
    compiler_params=pltpu.CompilerParams(use_tc_tiling_on_sc=False),
)(_sc_body)


def _eos_body(tok_ref, out_ref):
    x = tok_ref[...]
    m = jnp.max(x, axis=1, keepdims=True)
    ii = lax.broadcasted_iota(jnp.int32, x.shape, 1)
    first = jnp.min(jnp.where(x == m, ii, L), axis=1, keepdims=True)
    out_ref[...] = first + (1 + N_CTX)


_eos_call = pl.pallas_call(
    _eos_body,
    out_shape=jax.ShapeDtypeStruct((B, 1), jnp.int32),
)


@jax.jit
def kernel(tokens, table, ctx, prefix):
    tokens_i = tokens.astype(jnp.int32)
    prefix2d = prefix.reshape(1, D)
    prompts = _sc_call(tokens_i, table, ctx, prefix2d)
    eos = _eos_call(tokens_i).reshape(B)
    return (prompts, eos)


# speedup vs baseline: 2.8198x; 2.4330x over previous
import functools

import jax
import jax.numpy as jnp
from jax import lax
from jax.experimental import pallas as pl
from jax.experimental.pallas import tpu as pltpu
from jax.experimental.pallas import tpu_sc as plsc

N_CTX = 16
VOCAB = 49408
D = 512
B = 1024
L = 60
P = 1 + N_CTX + L
NHEAD = 1 + N_CTX

NC = 2
NS = 16
NW = NC * NS

CPR = D // 128
BH = B // 8
CPS = BH * 8 * CPR
HEAD_CHUNKS = NHEAD * CPS
GATH_CHUNKS = L * CPS
HPW = HEAD_CHUNKS // NW
GPW = GATH_CHUNKS // NW
HBS = 272
GBS = 384
NREP = 128


def _sc_body(head_hbm, chunks_hbm, idxh_hbm, idxg_hbm,
             out_hbm,
             idxh_v, idxg_v, buf0, buf1,
             gsem0, gsem1, wsem0, wsem1):
    wid = lax.axis_index("c") * NS + lax.axis_index("s")
    hs = wid * HPW
    gs = wid * GPW

    pltpu.sync_copy(idxh_hbm.at[pl.ds(hs, HPW)], idxh_v)
    pltpu.sync_copy(idxg_hbm.at[pl.ds(gs, GPW)], idxg_v)

    jobs = []
    for off in range(0, HPW, HBS):
        jobs.append((0, off, hs + off, HBS))
    for off in range(0, GPW, GBS):
        jobs.append((1, off, HEAD_CHUNKS + gs + off, GBS))
    n = len(jobs)

    bufs = (buf0, buf1)
    gsems = (gsem0, gsem1)
    wsems = (wsem0, wsem1)

    def start_gather(i, buf, sem):
        which, ioff, _, sz = jobs[i]
        src = head_hbm if which == 0 else chunks_hbm
        idxv = idxh_v if which == 0 else idxg_v
        return pltpu.async_copy(
            src.at[idxv.at[pl.ds(ioff, sz)]], buf.at[pl.ds(0, sz)], sem)

    gathers = [None] * n
    writes = [None] * n
    gathers[0] = start_gather(0, buf0, gsem0)
    for i in range(n):
        p = i % 2
        q = (i + 1) % 2
        if i + 1 < n:
            if i >= 1:
                writes[i - 1].wait()
            gathers[i + 1] = start_gather(i + 1, bufs[q], gsems[q])
        gathers[i].wait()
        _, _, ooff, sz = jobs[i]
        writes[i] = pltpu.async_copy(
            bufs[p].at[pl.ds(0, sz)], out_hbm.at[pl.ds(ooff, sz)], wsems[p])
    writes[n - 2].wait()
    writes[n - 1].wait()


_sc_call = functools.partial(
    pl.kernel,
    out_type=jax.ShapeDtypeStruct((HEAD_CHUNKS + GATH_CHUNKS, 128),
                                  jnp.float32),
    mesh=plsc.VectorSubcoreMesh(core_axis_name="c", subcore_axis_name="s"),
    scratch_types=[
        pltpu.VMEM((HPW,), jnp.int32),
        pltpu.VMEM((GPW,), jnp.int32),
        pltpu.VMEM((GBS, 128), jnp.float32),
        pltpu.VMEM((GBS, 128), jnp.float32),
        pltpu.SemaphoreType.DMA,
        pltpu.SemaphoreType.DMA,
        pltpu.SemaphoreType.DMA,
        pltpu.SemaphoreType.DMA,
    ],
    compiler_params=pltpu.CompilerParams(use_tc_tiling_on_sc=False),
)(_sc_body)


def _eos_body(tok_ref, out_ref):
    x = tok_ref[...]
    m = jnp.max(x, axis=1, keepdims=True)
    ii = lax.broadcasted_iota(jnp.int32, x.shape, 1)
    first = jnp.min(jnp.where(x == m, ii, L), axis=1, keepdims=True)
    out_ref[...] = first + NHEAD


_eos_call = pl.pallas_call(
    _eos_body,
    out_shape=jax.ShapeDtypeStruct((B, 1), jnp.int32),
)


@jax.jit
def kernel(tokens, table, ctx, prefix):
    tokens_i = tokens.astype(jnp.int32)

    chunks = (table.reshape(VOCAB // 8, 8, CPR, 128)
              .transpose(0, 2, 1, 3)
              .reshape(VOCAB * CPR, 128))

    head = jnp.concatenate([prefix.reshape(1, D), ctx], axis=0)
    head_chunks = head.reshape(NHEAD * CPR, 128)
    head_src = jnp.tile(head_chunks, (NREP, 1))

    t4 = jnp.arange(CPR, dtype=jnp.int32)
    bh = jnp.arange(BH, dtype=jnp.int32)
    pp = jnp.arange(NHEAD, dtype=jnp.int32)
    idx_head = (bh[None, :, None, None] * (NHEAD * CPR)
                + pp[:, None, None, None] * CPR
                + t4[None, None, :, None]
                + jnp.zeros((1, 1, 1, 8), jnp.int32)).reshape(-1)

    tok_r = tokens_i.reshape(BH, 8, L)
    base = (tok_r // 8) * 32 + tok_r % 8
    a2 = base.transpose(2, 0, 1)
    idx_gath = (a2[:, :, None, :]
                + (t4 * 8)[None, None, :, None]).reshape(-1)

    out_chunks = _sc_call(head_src, chunks, idx_head, idx_gath)
    prompts = (out_chunks.reshape(P, BH, CPR, 8, 128)
               .transpose(1, 3, 0, 2, 4)
               .reshape(B, P, D))
    eos = _eos_call(tokens_i).reshape(B)
    return (prompts, eos)


# speedup vs baseline: 2.8509x; 1.0110x over previous
import functools

import jax
import jax.numpy as jnp
from jax import lax
from jax.experimental import pallas as pl
from jax.experimental.pallas import tpu as pltpu
from jax.experimental.pallas import tpu_sc as plsc

N_CTX = 16
VOCAB = 49408
D = 512
B = 1024
L = 60
P = 1 + N_CTX + L
NHEAD = 1 + N_CTX

NC = 2
NS = 16
NW = NC * NS

CPR = D // 128
BH = B // 8
CPS = BH * 8 * CPR
HEAD_CHUNKS = NHEAD * CPS
GATH_CHUNKS = L * CPS
HPW = HEAD_CHUNKS // NW
GPW = GATH_CHUNKS // NW
HBS = 272
GBS = 256
NBUF = 3
NREP = 128


def _sc_body(head_hbm, chunks_hbm, idxh_hbm, idxg_hbm,
             out_hbm,
             idxh_v, idxg_v, buf0, buf1, buf2,
             gsem0, gsem1, gsem2, wsem0, wsem1, wsem2):
    wid = lax.axis_index("c") * NS + lax.axis_index("s")
    hs = wid * HPW
    gs = wid * GPW

    pltpu.sync_copy(idxh_hbm.at[pl.ds(hs, HPW)], idxh_v)
    pltpu.sync_copy(idxg_hbm.at[pl.ds(gs, GPW)], idxg_v)

    jobs = []
    for off in range(0, HPW, HBS):
        jobs.append((0, off, hs + off, HBS))
    for off in range(0, GPW, GBS):
        jobs.append((1, off, HEAD_CHUNKS + gs + off, GBS))
    n = len(jobs)

    bufs = (buf0, buf1, buf2)
    gsems = (gsem0, gsem1, gsem2)
    wsems = (wsem0, wsem1, wsem2)

    def start_gather(i, buf, sem):
        which, ioff, _, sz = jobs[i]
        src = head_hbm if which == 0 else chunks_hbm
        idxv = idxh_v if which == 0 else idxg_v
        return pltpu.async_copy(
            src.at[idxv.at[pl.ds(ioff, sz)]], buf.at[pl.ds(0, sz)], sem)

    gathers = [None] * n
    writes = [None] * n
    gathers[0] = start_gather(0, bufs[0], gsems[0])
    gathers[1] = start_gather(1, bufs[1], gsems[1])
    for i in range(n):
        p = i % NBUF
        if i + 2 < n:
            if i >= 1:
                writes[i - 1].wait()
            gathers[i + 2] = start_gather(
                i + 2, bufs[(i + 2) % NBUF], gsems[(i + 2) % NBUF])
        gathers[i].wait()
        _, _, ooff, sz = jobs[i]
        writes[i] = pltpu.async_copy(
            bufs[p].at[pl.ds(0, sz)], out_hbm.at[pl.ds(ooff, sz)], wsems[p])
    for j in range(max(0, n - NBUF), n):
        writes[j].wait()


_sc_call = functools.partial(
    pl.kernel,
    out_type=jax.ShapeDtypeStruct((HEAD_CHUNKS + GATH_CHUNKS, 128),
                                  jnp.float32),
    mesh=plsc.VectorSubcoreMesh(core_axis_name="c", subcore_axis_name="s"),
    scratch_types=[
        pltpu.VMEM((HPW,), jnp.int32),
        pltpu.VMEM((GPW,), jnp.int32),
        pltpu.VMEM((HBS, 128), jnp.float32),
        pltpu.VMEM((HBS, 128), jnp.float32),
        pltpu.VMEM((HBS, 128), jnp.float32),
        pltpu.SemaphoreType.DMA,
        pltpu.SemaphoreType.DMA,
        pltpu.SemaphoreType.DMA,
        pltpu.SemaphoreType.DMA,
        pltpu.SemaphoreType.DMA,
        pltpu.SemaphoreType.DMA,
    ],
    compiler_params=pltpu.CompilerParams(use_tc_tiling_on_sc=False),
)(_sc_body)


def _eos_body(tok_ref, out_ref):
    x = tok_ref[...]
    m = jnp.max(x, axis=1, keepdims=True)
    ii = lax.broadcasted_iota(jnp.int32, x.shape, 1)
    first = jnp.min(jnp.where(x == m, ii, L), axis=1, keepdims=True)
    out_ref[...] = first + NHEAD


_eos_call = pl.pallas_call(
    _eos_body,
    out_shape=jax.ShapeDtypeStruct((B, 1), jnp.int32),
)


@jax.jit
def kernel(tokens, table, ctx, prefix):
    tokens_i = tokens.astype(jnp.int32)

    chunks = (table.reshape(VOCAB // 8, 8, CPR, 128)
              .transpose(0, 2, 1, 3)
              .reshape(VOCAB * CPR, 128))

    head = jnp.concatenate([prefix.reshape(1, D), ctx], axis=0)
    head_chunks = head.reshape(NHEAD * CPR, 128)
    head_src = jnp.tile(head_chunks, (NREP, 1))

    t4 = jnp.arange(CPR, dtype=jnp.int32)
    bh = jnp.arange(BH, dtype=jnp.int32)
    pp = jnp.arange(NHEAD, dtype=jnp.int32)
    idx_head = (bh[None, :, None, None] * (NHEAD * CPR)
                + pp[:, None, None, None] * CPR
                + t4[None, None, :, None]
                + jnp.zeros((1, 1, 1, 8), jnp.int32)).reshape(-1)

    tok_r = tokens_i.reshape(BH, 8, L)
    base = (tok_r // 8) * 32 + tok_r % 8
    a2 = base.transpose(2, 0, 1)
    idx_gath = (a2[:, :, None, :]
                + (t4 * 8)[None, None, :, None]).reshape(-1)

    out_chunks = _sc_call(head_src, chunks, idx_head, idx_gath)
    prompts = (out_chunks.reshape(P, BH, CPR, 8, 128)
               .transpose(1, 3, 0, 2, 4)
               .reshape(B, P, D))
    eos = _eos_call(tokens_i).reshape(B)
    return (prompts, eos)


# speedup vs baseline: 3.4656x; 1.2156x over previous
import functools

import jax
import jax.numpy as jnp
from jax import lax
from jax.experimental import pallas as pl
from jax.experimental.pallas import tpu as pltpu
from jax.experimental.pallas import tpu_sc as plsc

N_CTX = 16
VOCAB = 49408
D = 512
B = 1024
L = 60
P = 1 + N_CTX + L
NHEAD = 1 + N_CTX

NC = 2
NS = 16
NW = NC * NS

CPR = D // 128
BH = B // 8
CPS = BH * 8 * CPR
HEAD_CHUNKS = NHEAD * CPS
GATH_CHUNKS = L * CPS
HPW = HEAD_CHUNKS // NW
GPW = GATH_CHUNKS // NW
GBS = 256
NBUF = 3
NREP = NW
GRP = 32


def _sc_body(head_hbm, chunks_hbm, idxt_hbm, idxg_hbm,
             out_hbm,
             idxt_v, idxg_v, tmpl_v, buf0, buf1, buf2,
             tsem, hwsem, gsem0, gsem1, gsem2, wsem0, wsem1, wsem2):
    wid = lax.axis_index("c") * NS + lax.axis_index("s")
    hs = wid * HPW
    gs = wid * GPW

    pltpu.sync_copy(idxt_hbm.at[pl.ds(wid * 2 * GRP, 2 * GRP)], idxt_v)
    pltpu.sync_copy(idxg_hbm.at[pl.ds(gs, GPW)], idxg_v)

    pltpu.async_copy(
        head_hbm.at[idxt_v.at[pl.ds(0, 2 * GRP)]], tmpl_v, tsem).wait()

    p0 = hs // CPS
    hwrites = []
    for off in range(0, HPW, GRP):
        sel = (hs + off) // CPS - p0
        hwrites.append(pltpu.async_copy(
            tmpl_v.at[pl.ds(sel * GRP, GRP)],
            out_hbm.at[pl.ds(hs + off, GRP)], hwsem))
        if len(hwrites) >= 9:
            hwrites[len(hwrites) - 9].wait()

    jobs = []
    for off in range(0, GPW, GBS):
        jobs.append((off, HEAD_CHUNKS + gs + off, GBS))
    n = len(jobs)

    bufs = (buf0, buf1, buf2)
    gsems = (gsem0, gsem1, gsem2)
    wsems = (wsem0, wsem1, wsem2)

    def start_gather(i, buf, sem):
        ioff, _, sz = jobs[i]
        return pltpu.async_copy(
            chunks_hbm.at[idxg_v.at[pl.ds(ioff, sz)]],
            buf.at[pl.ds(0, sz)], sem)

    gathers = [None] * n
    writes = [None] * n
    gathers[0] = start_gather(0, bufs[0], gsems[0])
    gathers[1] = start_gather(1, bufs[1], gsems[1])
    for i in range(n):
        p = i % NBUF
        if i + 2 < n:
            if i >= 1:
                writes[i - 1].wait()
            gathers[i + 2] = start_gather(
                i + 2, bufs[(i + 2) % NBUF], gsems[(i + 2) % NBUF])
        gathers[i].wait()
        _, ooff, sz = jobs[i]
        writes[i] = pltpu.async_copy(
            bufs[p].at[pl.ds(0, sz)], out_hbm.at[pl.ds(ooff, sz)], wsems[p])
    for j in range(max(0, n - NBUF), n):
        writes[j].wait()
    for h in hwrites[len(hwrites) - 8:]:
        h.wait()


_sc_call = functools.partial(
    pl.kernel,
    out_type=jax.ShapeDtypeStruct((HEAD_CHUNKS + GATH_CHUNKS, 128),
                                  jnp.float32),
    mesh=plsc.VectorSubcoreMesh(core_axis_name="c", subcore_axis_name="s"),
    scratch_types=[
        pltpu.VMEM((2 * GRP,), jnp.int32),
        pltpu.VMEM((GPW,), jnp.int32),
        pltpu.VMEM((2 * GRP, 128), jnp.float32),
        pltpu.VMEM((GBS, 128), jnp.float32),
        pltpu.VMEM((GBS, 128), jnp.float32),
        pltpu.VMEM((GBS, 128), jnp.float32),
        pltpu.SemaphoreType.DMA,
        pltpu.SemaphoreType.DMA,
        pltpu.SemaphoreType.DMA,
        pltpu.SemaphoreType.DMA,
        pltpu.SemaphoreType.DMA,
        pltpu.SemaphoreType.DMA,
        pltpu.SemaphoreType.DMA,
        pltpu.SemaphoreType.DMA,
    ],
    compiler_params=pltpu.CompilerParams(use_tc_tiling_on_sc=False),
)(_sc_body)


def _eos_body(tok_ref, out_ref):
    x = tok_ref[...]
    m = jnp.max(x, axis=1, keepdims=True)
    ii = lax.broadcasted_iota(jnp.int32, x.shape, 1)
    first = jnp.min(jnp.where(x == m, ii, L), axis=1, keepdims=True)
    out_ref[...] = first + NHEAD


_eos_call = pl.pallas_call(
    _eos_body,
    out_shape=jax.ShapeDtypeStruct((B, 1), jnp.int32),
)


@jax.jit
def kernel(tokens, table, ctx, prefix):
    tokens_i = tokens.astype(jnp.int32)

    chunks = (table.reshape(VOCAB // 8, 8, CPR, 128)
              .transpose(0, 2, 1, 3)
              .reshape(VOCAB * CPR, 128))

    head = jnp.concatenate([prefix.reshape(1, D), ctx], axis=0)
    head_chunks = head.reshape(NHEAD * CPR, 128)
    head_src = jnp.tile(head_chunks, (NREP, 1))

    t4 = jnp.arange(CPR, dtype=jnp.int32)
    ww = jnp.arange(NW, dtype=jnp.int32)
    jj = jnp.arange(2 * GRP, dtype=jnp.int32)
    p_t = jnp.minimum((ww * HPW)[:, None] // CPS + jj[None, :] // GRP,
                      NHEAD - 1)
    idx_tmpl = (ww[:, None] * (NHEAD * CPR) + p_t * CPR
                + ((jj % GRP) // 8)[None, :]).reshape(-1)

    tok_r = tokens_i.reshape(BH, 8, L)
    base = (tok_r // 8) * 32 + tok_r % 8
    a2 = base.transpose(2, 0, 1)
    idx_gath = (a2[:, :, None, :]
                + (t4 * 8)[None, None, :, None]).reshape(-1)

    out_chunks = _sc_call(head_src, chunks, idx_tmpl, idx_gath)
    prompts = (out_chunks.reshape(P, BH, CPR, 8, 128)
               .transpose(1, 3, 0, 2, 4)
               .reshape(B, P, D))
    eos = _eos_call(tokens_i).reshape(B)
    return (prompts, eos)


# speedup vs baseline: 3.4870x; 1.0062x over previous
import functools

import jax
import jax.numpy as jnp
from jax import lax
from jax.experimental import pallas as pl
from jax.experimental.pallas import tpu as pltpu
from jax.experimental.pallas import tpu_sc as plsc

N_CTX = 16
VOCAB = 49408
D = 512
B = 1024
L = 60
P = 1 + N_CTX + L
NHEAD = 1 + N_CTX

NC = 2
NS = 16
NW = NC * NS

CPR = D // 128
BH = B // 8
CPS = BH * 8 * CPR
HEAD_CHUNKS = NHEAD * CPS
GATH_CHUNKS = L * CPS
HPW = HEAD_CHUNKS // NW
GPW = GATH_CHUNKS // NW
GBS = 256
NBUF = 3
NREP = NW
GRP = 32


def _sc_body(head_hbm, chunks_hbm, idxt_hbm, idxg_hbm,
             out_hbm,
             idxt_v, idxg_v, tmpl_v, buf0, buf1, buf2,
             tsem, hwsem, gsem0, gsem1, gsem2, wsem0, wsem1, wsem2):
    wid = lax.axis_index("c") * NS + lax.axis_index("s")
    hs = wid * HPW
    gs = wid * GPW

    pltpu.sync_copy(idxt_hbm.at[pl.ds(wid * 2 * GRP, 2 * GRP)], idxt_v)
    idxg_stage = pltpu.async_copy(idxg_hbm.at[pl.ds(gs, GPW)], idxg_v, tsem)

    pltpu.async_copy(
        head_hbm.at[idxt_v.at[pl.ds(0, 2 * GRP)]], tmpl_v, hwsem).wait()

    p0 = hs // CPS
    hwrites = []
    for off in range(0, HPW, GRP):
        sel = (hs + off) // CPS - p0
        hwrites.append(pltpu.async_copy(
            tmpl_v.at[pl.ds(sel * GRP, GRP)],
            out_hbm.at[pl.ds(hs + off, GRP)], hwsem))
        if len(hwrites) >= 9:
            hwrites[len(hwrites) - 9].wait()

    idxg_stage.wait()

    jobs = []
    for off in range(0, GPW, GBS):
        jobs.append((off, HEAD_CHUNKS + gs + off, GBS))
    n = len(jobs)

    bufs = (buf0, buf1, buf2)
    gsems = (gsem0, gsem1, gsem2)
    wsems = (wsem0, wsem1, wsem2)

    def start_gather(i, buf, sem):
        ioff, _, sz = jobs[i]
        return pltpu.async_copy(
            chunks_hbm.at[idxg_v.at[pl.ds(ioff, sz)]],
            buf.at[pl.ds(0, sz)], sem)

    gathers = [None] * n
    writes = [None] * n
    gathers[0] = start_gather(0, bufs[0], gsems[0])
    gathers[1] = start_gather(1, bufs[1], gsems[1])
    for i in range(n):
        p = i % NBUF
        if i + 2 < n:
            if i >= 1:
                writes[i - 1].wait()
            gathers[i + 2] = start_gather(
                i + 2, bufs[(i + 2) % NBUF], gsems[(i + 2) % NBUF])
        gathers[i].wait()
        _, ooff, sz = jobs[i]
        writes[i] = pltpu.async_copy(
            bufs[p].at[pl.ds(0, sz)], out_hbm.at[pl.ds(ooff, sz)], wsems[p])
    for j in range(max(0, n - NBUF), n):
        writes[j].wait()
    for h in hwrites[len(hwrites) - 8:]:
        h.wait()


_sc_call = functools.partial(
    pl.kernel,
    out_type=jax.ShapeDtypeStruct((HEAD_CHUNKS + GATH_CHUNKS, 128),
                                  jnp.float32),
    mesh=plsc.VectorSubcoreMesh(core_axis_name="c", subcore_axis_name="s"),
    scratch_types=[
        pltpu.VMEM((2 * GRP,), jnp.int32),
        pltpu.VMEM((GPW,), jnp.int32),
        pltpu.VMEM((2 * GRP, 128), jnp.float32),
        pltpu.VMEM((GBS, 128), jnp.float32),
        pltpu.VMEM((GBS, 128), jnp.float32),
        pltpu.VMEM((GBS, 128), jnp.float32),
        pltpu.SemaphoreType.DMA,
        pltpu.SemaphoreType.DMA,
        pltpu.SemaphoreType.DMA,
        pltpu.SemaphoreType.DMA,
        pltpu.SemaphoreType.DMA,
        pltpu.SemaphoreType.DMA,
        pltpu.SemaphoreType.DMA,
        pltpu.SemaphoreType.DMA,
    ],
    compiler_params=pltpu.CompilerParams(use_tc_tiling_on_sc=False),
)(_sc_body)


def _eos_body(tok_ref, out_ref):
    x = tok_ref[...]
    m = jnp.max(x, axis=1, keepdims=True)
    ii = lax.broadcasted_iota(jnp.int32, x.shape, 1)
    first = jnp.min(jnp.where(x == m, ii, L), axis=1, keepdims=True)
    out_ref[...] = first + NHEAD


_eos_call = pl.pallas_call(
    _eos_body,
    out_shape=jax.ShapeDtypeStruct((B, 1), jnp.int32),
)


@jax.jit
def kernel(tokens, table, ctx, prefix):
    tokens_i = tokens.astype(jnp.int32)

    chunks = (table.reshape(VOCAB // 8, 8, CPR, 128)
              .transpose(0, 2, 1, 3)
              .reshape(VOCAB * CPR, 128))

    head = jnp.concatenate([prefix.reshape(1, D), ctx], axis=0)
    head_chunks = head.reshape(NHEAD * CPR, 128)
    head_src = jnp.tile(head_chunks, (NREP, 1))

    t4 = jnp.arange(CPR, dtype=jnp.int32)
    ww = jnp.arange(NW, dtype=jnp.int32)
    jj = jnp.arange(2 * GRP, dtype=jnp.int32)
    p_t = jnp.minimum((ww * HPW)[:, None] // CPS + jj[None, :] // GRP,
                      NHEAD - 1)
    idx_tmpl = (ww[:, None] * (NHEAD * CPR) + p_t * CPR
                + ((jj % GRP) // 8)[None, :]).reshape(-1)

    tok_r = tokens_i.reshape(BH, 8, L)
    base = (tok_r // 8) * 32 + tok_r % 8
    a2 = base.transpose(2, 0, 1)
    idx_gath = (a2[:, :, None, :]
                + (t4 * 8)[None, None, :, None]).reshape(-1)

    out_chunks = _sc_call(head_src, chunks, idx_tmpl, idx_gath)
    prompts = (out_chunks.reshape(P, BH, CPR, 8, 128)
               .transpose(1, 3, 0, 2, 4)
               .reshape(B, P, D))
    eos = _eos_call(tokens_i).reshape(B)
    return (prompts, eos)
